# baseline (speedup 1.0000x reference)
import dataclasses
import functools

import jax
import jax.numpy as jnp
from jax import lax
from jax.experimental import pallas as pl
from jax.experimental.pallas import tpu as pltpu
from jax.experimental.pallas import tpu_sc as plsc

N = 10000
K = 16
L = 16
NC = 2
NS = 16
NW = NC * NS
NPW = 312
NTAIL = N - NW * NPW


def _arap_tec(pc_hbm, idx_hbm, dist_hbm, out_hbm, pc_v, idx_v, dist_v, acc_v):
    cid = lax.axis_index("c")
    sid = lax.axis_index("s")
    wid = sid * NC + cid
    start = wid * NPW

    pltpu.sync_copy(pc_hbm, pc_v)
    pltpu.sync_copy(idx_hbm.at[pl.ds(start, NPW)], idx_v.at[pl.ds(0, NPW)])
    pltpu.sync_copy(dist_hbm.at[pl.ds(start, NPW)], dist_v.at[pl.ds(0, NPW)])
    pltpu.sync_copy(idx_hbm.at[pl.ds(NW * NPW, NTAIL)],
                    idx_v.at[pl.ds(NPW, NTAIL)])
    pltpu.sync_copy(dist_hbm.at[pl.ds(NW * NPW, NTAIL)],
                    dist_v.at[pl.ds(NPW, NTAIL)])

    col0 = jnp.zeros((L,), jnp.int32)
    col1 = col0 + 1
    col2 = col0 + 2

    def point_term(t, i):
        idx_row = idx_v[t, :]
        gx = plsc.load_gather(pc_v, [idx_row, col0])
        gy = plsc.load_gather(pc_v, [idx_row, col1])
        gz = plsc.load_gather(pc_v, [idx_row, col2])
        ivec = jnp.full((L,), i, jnp.int32)
        cx = plsc.load_gather(pc_v, [ivec, col0])
        cy = plsc.load_gather(pc_v, [ivec, col1])
        cz = plsc.load_gather(pc_v, [ivec, col2])
        dx = cx - gx
        dy = cy - gy
        dz = cz - gz
        d2 = dx * dx + dy * dy + dz * dz
        return jnp.abs(d2 - dist_v[t, :])

    def body(t, acc):
        return acc + point_term(t, start + t)

    acc = lax.fori_loop(0, NPW, body, jnp.zeros((L,), jnp.float32))
    tail_term = point_term(NPW + jnp.minimum(wid, NTAIL - 1),
                           NW * NPW + jnp.minimum(wid, NTAIL - 1))
    acc = acc + jnp.where(wid < NTAIL, tail_term, 0.0)
    acc_v[...] = acc
    pltpu.sync_copy(acc_v, out_hbm.at[wid])


@jax.jit
def _arap_sc(pc, idx, dist):
    cp = pltpu.CompilerParams(use_tc_tiling_on_sc=False)
    if "needs_layout_passes" in pltpu.CompilerParams.__dataclass_fields__:
        cp = dataclasses.replace(cp, needs_layout_passes=False)
    run = pl.kernel(
        _arap_tec,
        out_type=jax.ShapeDtypeStruct((NW, L), jnp.float32),
        compiler_params=cp,
        mesh=plsc.VectorSubcoreMesh(core_axis_name="c", subcore_axis_name="s"),
        scratch_types=[
            pltpu.VMEM((N, 3), jnp.float32),
            pltpu.VMEM((NPW + NTAIL, K), jnp.int32),
            pltpu.VMEM((NPW + NTAIL, K), jnp.float32),
            pltpu.VMEM((L,), jnp.float32),
        ],
    )
    return run(pc, idx, dist)


def kernel(pc_transformed, nn_distances, nn_indices):
    if nn_indices.dtype != jnp.int32:
        nn_indices = nn_indices.astype(jnp.int32)
    partials = _arap_sc(pc_transformed, nn_indices, nn_distances)
    return jnp.sum(partials) / (N * K)


# speedup vs baseline: 1.5767x; 1.5767x over previous
import dataclasses

import jax
import jax.numpy as jnp
from jax import lax
from jax.experimental import pallas as pl
from jax.experimental.pallas import tpu as pltpu
from jax.experimental.pallas import tpu_sc as plsc

N = 10000
K = 16
L = 16
NC = 2
NS = 16
NW = NC * NS
NPW = 320
NPAD = NW * NPW
NB = NPW // L


def _arap_tec(planes_hbm, bidx_hbm, bdist_hbm, out_hbm,
              pcx_v, pcy_v, pcz_v, bidx_v, bdist_v, acc_v):
    cid = lax.axis_index("c")
    sid = lax.axis_index("s")
    wid = sid * NC + cid
    base = wid * NPW

    pltpu.sync_copy(planes_hbm.at[0], pcx_v)
    pltpu.sync_copy(planes_hbm.at[1], pcy_v)
    pltpu.sync_copy(planes_hbm.at[2], pcz_v)
    pltpu.sync_copy(bidx_hbm.at[wid], bidx_v)
    pltpu.sync_copy(bdist_hbm.at[wid], bdist_v)

    def outer(b, accs):
        acc0, acc1 = accs
        p0 = b * L
        cx = pcx_v[pl.ds(base + p0, L)]
        cy = pcy_v[pl.ds(base + p0, L)]
        cz = pcz_v[pl.ds(base + p0, L)]
        for k in range(K):
            idxv = bidx_v[k, pl.ds(p0, L)]
            gx = plsc.load_gather(pcx_v, [idxv])
            gy = plsc.load_gather(pcy_v, [idxv])
            gz = plsc.load_gather(pcz_v, [idxv])
            dx = cx - gx
            dy = cy - gy
            dz = cz - gz
            d2 = dx * dx + dy * dy + dz * dz
            term = jnp.abs(d2 - bdist_v[k, pl.ds(p0, L)])
            if k % 2 == 0:
                acc0 = acc0 + term
            else:
                acc1 = acc1 + term
        return acc0, acc1

    zero = jnp.zeros((L,), jnp.float32)
    acc0, acc1 = lax.fori_loop(0, NB, outer, (zero, zero))
    acc_v[...] = acc0 + acc1
    pltpu.sync_copy(acc_v, out_hbm.at[wid])


@jax.jit
def _arap_sc(planes, bidx, bdist):
    cp = pltpu.CompilerParams(use_tc_tiling_on_sc=False)
    if "needs_layout_passes" in pltpu.CompilerParams.__dataclass_fields__:
        cp = dataclasses.replace(cp, needs_layout_passes=False)
    run = pl.kernel(
        _arap_tec,
        out_type=jax.ShapeDtypeStruct((NW, L), jnp.float32),
        compiler_params=cp,
        mesh=plsc.VectorSubcoreMesh(core_axis_name="c", subcore_axis_name="s"),
        scratch_types=[
            pltpu.VMEM((NPAD + L,), jnp.float32),
            pltpu.VMEM((NPAD + L,), jnp.float32),
            pltpu.VMEM((NPAD + L,), jnp.float32),
            pltpu.VMEM((K, NPW), jnp.int32),
            pltpu.VMEM((K, NPW), jnp.float32),
            pltpu.VMEM((L,), jnp.float32),
        ],
    )
    return run(planes, bidx, bdist)


def kernel(pc_transformed, nn_distances, nn_indices):
    if nn_indices.dtype != jnp.int32:
        nn_indices = nn_indices.astype(jnp.int32)
    idxp = jnp.pad(nn_indices, ((0, NPAD - N), (0, 0)), constant_values=N)
    distp = jnp.pad(nn_distances, ((0, NPAD - N), (0, 0)))
    bidx = idxp.T.reshape(K, NW, NPW).transpose(1, 0, 2)
    bdist = distp.T.reshape(K, NW, NPW).transpose(1, 0, 2)
    planes = jnp.pad(pc_transformed.T, ((0, 0), (0, NPAD - N + L)))
    partials = _arap_sc(planes, bidx, bdist)
    return jnp.sum(partials) / (N * K)


# speedup vs baseline: 1.6336x; 1.0361x over previous
import dataclasses

import jax
import jax.numpy as jnp
from jax import lax
from jax.experimental import pallas as pl
from jax.experimental.pallas import tpu as pltpu
from jax.experimental.pallas import tpu_sc as plsc

N = 10000
K = 16
L = 16
NC = 2
NS = 16
NW = NC * NS
NPW = 320
NB = NPW // L


def _arap_tec(pc_hbm, idx_hbm, dist_hbm, out_hbm,
              pcx_v, pcy_v, pcz_v, bidx_v, bdist_v, acc_v):
    cid = lax.axis_index("c")
    sid = lax.axis_index("s")
    wid = sid * NC + cid
    start = jnp.minimum(wid * NPW, N - NPW)
    b_lo = jnp.where(wid == NW - 1, NB - (N - (NW - 1) * NPW) // L, 0)

    pltpu.sync_copy(pc_hbm.at[0], pcx_v)
    pltpu.sync_copy(pc_hbm.at[1], pcy_v)
    pltpu.sync_copy(pc_hbm.at[2], pcz_v)
    pltpu.sync_copy(idx_hbm.at[:, pl.ds(start, NPW)], bidx_v)
    pltpu.sync_copy(dist_hbm.at[:, pl.ds(start, NPW)], bdist_v)

    def outer(b, accs):
        acc0, acc1 = accs
        p0 = b * L
        cx = pcx_v[pl.ds(start + p0, L)]
        cy = pcy_v[pl.ds(start + p0, L)]
        cz = pcz_v[pl.ds(start + p0, L)]
        for k in range(K):
            idxv = bidx_v[k, pl.ds(p0, L)]
            gx = plsc.load_gather(pcx_v, [idxv])
            gy = plsc.load_gather(pcy_v, [idxv])
            gz = plsc.load_gather(pcz_v, [idxv])
            dx = cx - gx
            dy = cy - gy
            dz = cz - gz
            d2 = dx * dx + dy * dy + dz * dz
            term = jnp.abs(d2 - bdist_v[k, pl.ds(p0, L)])
            if k % 2 == 0:
                acc0 = acc0 + term
            else:
                acc1 = acc1 + term
        return acc0, acc1

    zero = jnp.zeros((L,), jnp.float32)
    acc0, acc1 = lax.fori_loop(b_lo, NB, outer, (zero, zero))
    acc_v[...] = acc0 + acc1
    pltpu.sync_copy(acc_v, out_hbm.at[wid])


@jax.jit
def _arap_sc(pcT, idxT, distT):
    cp = pltpu.CompilerParams(use_tc_tiling_on_sc=False)
    if "needs_layout_passes" in pltpu.CompilerParams.__dataclass_fields__:
        cp = dataclasses.replace(cp, needs_layout_passes=False)
    run = pl.kernel(
        _arap_tec,
        out_type=jax.ShapeDtypeStruct((NW, L), jnp.float32),
        compiler_params=cp,
        mesh=plsc.VectorSubcoreMesh(core_axis_name="c", subcore_axis_name="s"),
        scratch_types=[
            pltpu.VMEM((N,), jnp.float32),
            pltpu.VMEM((N,), jnp.float32),
            pltpu.VMEM((N,), jnp.float32),
            pltpu.VMEM((K, NPW), jnp.int32),
            pltpu.VMEM((K, NPW), jnp.float32),
            pltpu.VMEM((L,), jnp.float32),
        ],
    )
    return run(pcT, idxT, distT)


def kernel(pc_transformed, nn_distances, nn_indices):
    if nn_indices.dtype != jnp.int32:
        nn_indices = nn_indices.astype(jnp.int32)
    partials = _arap_sc(pc_transformed.T, nn_indices.T, nn_distances.T)
    return jnp.sum(partials) / (N * K)
